# Initial kernel scaffold; baseline (speedup 1.0000x reference)
import jax
import jax.numpy as jnp
from jax.experimental import pallas as pl


def kernel(rad_basis, sph_basis, idx_sph_outer, weight):
    raise NotImplementedError("write your pallas kernel here")


# speedup vs baseline: 3.3227x; 3.3227x over previous
import functools

import jax
import jax.numpy as jnp
from jax import lax
from jax.experimental import pallas as pl
from jax.experimental.pallas import tpu as pltpu
from jax.experimental.pallas import tpu_sc as plsc

_NW = 32
_GROW = 128
_MACRO = 8
_HALF = 4


def _sc_gather(rad_basis, idx2d):
    n_rows = idx2d.shape[0]
    feat = rad_basis.shape[1]
    total = n_rows * _GROW
    n_macro = n_rows // _MACRO
    mesh = plsc.VectorSubcoreMesh(core_axis_name="c", subcore_axis_name="s")

    @functools.partial(
        pl.kernel,
        mesh=mesh,
        out_type=jax.ShapeDtypeStruct((total, feat), jnp.float32),
        scratch_types=[
            pltpu.VMEM((_MACRO, _GROW), jnp.int32),
            pltpu.VMEM((_HALF * _GROW, feat), jnp.float32),
            pltpu.SemaphoreType.DMA,
        ],
    )
    def k(rad_hbm, idx_hbm, out_hbm, idx_v, rows_v, sem):
        c = lax.axis_index("c")
        s = lax.axis_index("s")
        wid = s * 2 + c
        m0 = (n_macro * wid) // _NW
        m1 = (n_macro * (wid + 1)) // _NW

        def body(m, carry):
            rbase = m * _MACRO
            pltpu.sync_copy(idx_hbm.at[pl.ds(rbase, _MACRO)], idx_v)
            for h in range(_MACRO // _HALF):
                cps = [
                    pltpu.async_copy(
                        rad_hbm.at[idx_v.at[h * _HALF + j]],
                        rows_v.at[pl.ds(j * _GROW, _GROW)],
                        sem,
                    )
                    for j in range(_HALF)
                ]
                for cp in cps:
                    cp.wait()
                pltpu.sync_copy(
                    rows_v,
                    out_hbm.at[
                        pl.ds((rbase + h * _HALF) * _GROW, _HALF * _GROW)
                    ],
                )
            return carry

        lax.fori_loop(m0, m1, body, 0)

    return k(rad_basis, idx2d)


_TB = 512


def _tc_body(g_ref, sph_ref, w_ref, out_ref):
    h = jnp.dot(g_ref[...], w_ref[...], preferred_element_type=jnp.float32)
    sph = sph_ref[...]
    m = out_ref.shape[1]
    acc = h[:, 0:m] * sph[:, 0:1]
    for b in range(1, sph.shape[1]):
        acc = acc + h[:, b * m:(b + 1) * m] * sph[:, b:b + 1]
    out_ref[...] = acc


def _tc_contract(g, sph, w2b):
    t = g.shape[0]
    feat = g.shape[1]
    n = w2b.shape[1]
    m = n // sph.shape[1]
    return pl.pallas_call(
        _tc_body,
        grid=(t // _TB,),
        in_specs=[
            pl.BlockSpec((_TB, feat), lambda i: (i, 0)),
            pl.BlockSpec((_TB, sph.shape[1]), lambda i: (i, 0)),
            pl.BlockSpec(w2b.shape, lambda i: (0, 0)),
        ],
        out_specs=pl.BlockSpec((_TB, m), lambda i: (i, 0)),
        out_shape=jax.ShapeDtypeStruct((t, m), jnp.float32),
    )(g, sph, w2b)


def kernel(rad_basis, sph_basis, idx_sph_outer, weight):
    num_radial = weight.shape[0]
    num_sph = weight.shape[1]
    emb = weight.shape[2]
    idx2d = idx_sph_outer.astype(jnp.int32).reshape(-1, _GROW)
    g = _sc_gather(rad_basis, idx2d)
    w2b = (
        weight.reshape(num_radial, -1)
        .reshape(num_radial, emb, num_sph)
        .transpose(0, 2, 1)
        .reshape(num_radial, num_sph * emb)
    )
    return _tc_contract(g, sph_basis, w2b)
